# baseline (speedup 1.0000x reference)
import functools

import jax
import jax.numpy as jnp
from jax import lax
from jax.experimental import pallas as pl
from jax.experimental.pallas import tpu as pltpu
from jax.experimental.pallas import tpu_sc as plsc

BATCH = 4
SEQ_LEN = 2048
VOCAB = 50257
HIDDEN = 768

_info = plsc.get_sparse_core_info()
_NC = _info.num_cores
_NS = _info.num_subcores
NW = _NC * _NS
N = BATCH * SEQ_LEN
PER_W = N // NW
CHUNK = 32
NCHUNK = PER_W // CHUNK
NBUF = 5

_mesh = plsc.VectorSubcoreMesh(core_axis_name="c", subcore_axis_name="s")


W_PER_B = SEQ_LEN // PER_W


@functools.partial(
    pl.kernel,
    mesh=_mesh,
    out_type=jax.ShapeDtypeStruct((BATCH, SEQ_LEN, HIDDEN), jnp.float32),
    scratch_types=[
        pltpu.VMEM((PER_W,), jnp.int32),
        *[pltpu.VMEM((CHUNK, HIDDEN), jnp.float32) for _ in range(NBUF)],
        *[pltpu.SemaphoreType.DMA for _ in range(2 * NBUF)],
    ],
)
def _emb_lookup(ids_hbm, table_hbm, out_hbm, idx_v, *rest):
    bufs = rest[:NBUF]
    gsems = rest[NBUF : 2 * NBUF]
    ssems = rest[2 * NBUF : 3 * NBUF]
    wid = lax.axis_index("c") * _NS + lax.axis_index("s")
    row = wid // W_PER_B
    col = (wid % W_PER_B) * PER_W

    pltpu.sync_copy(ids_hbm.at[row, pl.ds(col, PER_W)], idx_v)

    def gather_args(c, b):
        return (
            table_hbm.at[idx_v.at[pl.ds(c * CHUNK, CHUNK)]],
            bufs[b],
            gsems[b],
        )

    def store_args(c, b):
        return (bufs[b], out_hbm.at[row, pl.ds(col + c * CHUNK, CHUNK)], ssems[b])

    LAG = 0
    for b in range(min(NBUF, NCHUNK)):
        pltpu.async_copy(*gather_args(b, b))
    for c in range(NCHUNK):
        pltpu.make_async_copy(*gather_args(c, c % NBUF)).wait()
        pltpu.async_copy(*store_args(c, c % NBUF))
        d = c - LAG
        if d >= 0 and d + NBUF < NCHUNK:
            pltpu.make_async_copy(*store_args(d, d % NBUF)).wait()
            pltpu.async_copy(*gather_args(d + NBUF, d % NBUF))
    for c in range(max(0, NCHUNK - NBUF), NCHUNK):
        pltpu.make_async_copy(*store_args(c, c % NBUF)).wait()


def kernel(input_ids, embedding_weight):
    return _emb_lookup(input_ids.astype(jnp.int32), embedding_weight)


# speedup vs baseline: 1.0081x; 1.0081x over previous
import functools

import jax
import jax.numpy as jnp
from jax import lax
from jax.experimental import pallas as pl
from jax.experimental.pallas import tpu as pltpu
from jax.experimental.pallas import tpu_sc as plsc

BATCH = 4
SEQ_LEN = 2048
VOCAB = 50257
HIDDEN = 768

_info = plsc.get_sparse_core_info()
_NC = _info.num_cores
_NS = _info.num_subcores
NW = _NC * _NS
N = BATCH * SEQ_LEN
PER_W = N // NW
CHUNK = 32
NCHUNK = PER_W // CHUNK
NBUF = 5

_mesh = plsc.VectorSubcoreMesh(core_axis_name="c", subcore_axis_name="s")


W_PER_B = SEQ_LEN // PER_W


@functools.partial(
    pl.kernel,
    mesh=_mesh,
    out_type=jax.ShapeDtypeStruct((BATCH, SEQ_LEN, HIDDEN), jnp.float32),
    scratch_types=[
        pltpu.VMEM((PER_W,), jnp.int32),
        *[pltpu.VMEM((CHUNK, HIDDEN), jnp.float32) for _ in range(NBUF)],
        *[pltpu.SemaphoreType.DMA for _ in range(2 * NBUF)],
    ],
)
def _emb_lookup(ids_hbm, table_hbm, out_hbm, idx_v, *rest):
    bufs = rest[:NBUF]
    gsems = rest[NBUF : 2 * NBUF]
    ssems = rest[2 * NBUF : 3 * NBUF]
    wid = lax.axis_index("c") * _NS + lax.axis_index("s")
    row = wid // W_PER_B
    col = (wid % W_PER_B) * PER_W

    def gather_args(c, b):
        return (
            table_hbm.at[idx_v.at[pl.ds(c * CHUNK, CHUNK)]],
            bufs[b],
            gsems[b],
        )

    def store_args(c, b):
        return (bufs[b], out_hbm.at[row, pl.ds(col + c * CHUNK, CHUNK)], ssems[b])

    LAG = 0
    HALF = PER_W // 2
    pltpu.sync_copy(ids_hbm.at[row, pl.ds(col, HALF)], idx_v.at[pl.ds(0, HALF)])
    nprime = min(NBUF, NCHUNK)
    for b in range(nprime):
        if (b + 1) * CHUNK <= HALF:
            pltpu.async_copy(*gather_args(b, b))
    pltpu.sync_copy(
        ids_hbm.at[row, pl.ds(col + HALF, PER_W - HALF)],
        idx_v.at[pl.ds(HALF, PER_W - HALF)],
    )
    for b in range(nprime):
        if (b + 1) * CHUNK > HALF:
            pltpu.async_copy(*gather_args(b, b))
    for c in range(NCHUNK):
        pltpu.make_async_copy(*gather_args(c, c % NBUF)).wait()
        pltpu.async_copy(*store_args(c, c % NBUF))
        d = c - LAG
        if d >= 0 and d + NBUF < NCHUNK:
            pltpu.make_async_copy(*store_args(d, d % NBUF)).wait()
            pltpu.async_copy(*gather_args(d + NBUF, d % NBUF))
    for c in range(max(0, NCHUNK - NBUF), NCHUNK):
        pltpu.make_async_copy(*store_args(c, c % NBUF)).wait()


def kernel(input_ids, embedding_weight):
    return _emb_lookup(input_ids.astype(jnp.int32), embedding_weight)
